# Initial kernel scaffold; baseline (speedup 1.0000x reference)
import jax
import jax.numpy as jnp
from jax.experimental import pallas as pl


def kernel(x, edge_index, edge_attrs, Wq, bq, Wk, bk, Wv, bv, We, Ws, bs, Wg, bg):
    raise NotImplementedError("write your pallas kernel here")


# speedup vs baseline: 3.5033x; 3.5033x over previous
import functools

import jax
import jax.numpy as jnp
from jax.experimental import pallas as pl

N = 10000
E = 320000
F = 128
IC = 32
NB = F // IC
HC = 32
OUT = 128


def _matmul_body(x_ref, w_ref, b_ref, o_ref):
    o_ref[...] = jnp.dot(x_ref[...], w_ref[...],
                         preferred_element_type=jnp.float32) + b_ref[...]


def _tc_matmul(x, w, b, block_rows):
    n, f = x.shape
    _, m = w.shape
    grid = (n // block_rows,)
    return pl.pallas_call(
        _matmul_body,
        grid=grid,
        in_specs=[
            pl.BlockSpec((block_rows, f), lambda i: (i, 0)),
            pl.BlockSpec((f, m), lambda i: (0, 0)),
            pl.BlockSpec((1, m), lambda i: (0, 0)),
        ],
        out_specs=pl.BlockSpec((block_rows, m), lambda i: (i, 0)),
        out_shape=jax.ShapeDtypeStruct((n, m), jnp.float32),
    )(x, w, b)


def _block_diag(w):
    out = jnp.zeros((F, NB * HC), dtype=w.dtype)
    for i in range(NB):
        out = out.at[i * IC:(i + 1) * IC, i * HC:(i + 1) * HC].set(w)
    return out


def kernel(x, edge_index, edge_attrs, Wq, bq, Wk, bk, Wv, bv, We, Ws, bs, Wg, bg):
    src = edge_index[0]
    dst = edge_index[1]

    Wcat = jnp.concatenate(
        [_block_diag(Wq), _block_diag(Wk), _block_diag(Wv), _block_diag(Ws)],
        axis=1)
    bcat = jnp.concatenate(
        [jnp.tile(bq, NB), jnp.tile(bk, NB), jnp.tile(bv, NB), jnp.tile(bs, NB)])
    QKVS = _tc_matmul(x, Wcat, bcat[None, :], block_rows=2000)
    Q, K, V, S = (QKVS[:, 0:F], QKVS[:, F:2 * F], QKVS[:, 2 * F:3 * F],
                  QKVS[:, 3 * F:4 * F])

    e = edge_attrs @ We
    er = jnp.tile(e, (1, NB))

    ke = K[src] + er
    alpha = (Q[dst] * ke).reshape(E, NB, HC).sum(-1) / jnp.sqrt(jnp.float32(HC))
    ex = jnp.exp(alpha)
    den = jax.ops.segment_sum(ex, dst, num_segments=N)
    msg = (V[src] + er) * jnp.repeat(ex, HC, axis=1)
    num = jax.ops.segment_sum(msg, dst, num_segments=N)
    h = jax.nn.relu(num / jnp.repeat(den + 1e-16, HC, axis=1) + S)

    ew = edge_attrs[:, 1]
    deg = jax.ops.segment_sum(ew, dst, num_segments=N) + 2.0
    dis = deg ** -0.5
    norm = dis[src] * ew * dis[dst]
    hw = _tc_matmul(h, Wg, bg[None, :] * 0.0, block_rows=2000)
    out = jax.ops.segment_sum(norm[:, None] * hw[src], num_segments=N,
                              segment_ids=dst)
    out = out + (2.0 * dis * dis)[:, None] * hw + bg
    return out


# speedup vs baseline: 13.4525x; 3.8400x over previous
import functools

import numpy as np
import jax
import jax.numpy as jnp
from jax import lax
from jax.experimental import pallas as pl
from jax.experimental.pallas import tpu as pltpu
from jax.experimental.pallas import tpu_sc as plsc

N = 10000
E = 320000
F = 128
IC = 32
NB = F // IC
HC = 32
OUT = 128

NC = 2
NS = 16
L = 16
NW = NC * NS
EPW = E // NW
CH = 40
NCHUNK = EPW // CH
NACC = 10240
RPS = NACC // NS

_INV_SQRT = 1.0 / float(np.sqrt(HC))


def _mesh():
    return plsc.VectorSubcoreMesh(core_axis_name="c", subcore_axis_name="s",
                                  num_cores=NC, num_subcores=NS)


def _proj_body(x_ref, w_ref, b_ref, q_ref, k_ref, v_ref, s_ref):
    r = jnp.dot(x_ref[...], w_ref[...],
                preferred_element_type=jnp.float32) + b_ref[...]
    q_ref[...] = r[:, 0:F]
    k_ref[...] = r[:, F:2 * F]
    v_ref[...] = r[:, 2 * F:3 * F]
    s_ref[...] = r[:, 3 * F:4 * F]


def _tc_proj(x, wcat, bcat, block_rows=2000):
    grid = (N // block_rows,)
    outs = [jax.ShapeDtypeStruct((N, F), jnp.float32)] * 4
    return pl.pallas_call(
        _proj_body,
        grid=grid,
        in_specs=[
            pl.BlockSpec((block_rows, F), lambda i: (i, 0)),
            pl.BlockSpec((F, 4 * F), lambda i: (0, 0)),
            pl.BlockSpec((1, 4 * F), lambda i: (0, 0)),
        ],
        out_specs=[pl.BlockSpec((block_rows, F), lambda i: (i, 0))] * 4,
        out_shape=outs,
    )(x, wcat, bcat)


def _mid_body(np_ref, dp_ref, s_ref, wg_ref, hw_ref, dis_ref):
    num = np_ref[0] + np_ref[1]
    den = dp_ref[0] + dp_ref[1]
    den4 = den[:, 0:NB] + 1e-16
    denr = jnp.concatenate(
        [jnp.broadcast_to(den4[:, b:b + 1], (num.shape[0], HC))
         for b in range(NB)], axis=1)
    h = jnp.maximum(num / denr + s_ref[...], 0.0)
    hw_ref[...] = jnp.dot(h, wg_ref[...], preferred_element_type=jnp.float32)
    deg = den[:, NB:NB + 1] + 2.0
    dis_ref[...] = lax.rsqrt(deg)


def _tc_mid(num_p, den_p, s, wg, block_rows=2000):
    grid = (N // block_rows,)
    return pl.pallas_call(
        _mid_body,
        grid=grid,
        in_specs=[
            pl.BlockSpec((NC, block_rows, F), lambda i: (0, i, 0)),
            pl.BlockSpec((NC, block_rows, 16), lambda i: (0, i, 0)),
            pl.BlockSpec((block_rows, F), lambda i: (i, 0)),
            pl.BlockSpec((F, OUT), lambda i: (0, 0)),
        ],
        out_specs=[
            pl.BlockSpec((block_rows, OUT), lambda i: (i, 0)),
            pl.BlockSpec((block_rows, 1), lambda i: (i, 0)),
        ],
        out_shape=[
            jax.ShapeDtypeStruct((N, OUT), jnp.float32),
            jax.ShapeDtypeStruct((N, 1), jnp.float32),
        ],
    )(num_p, den_p, s, wg)


def _fin_body(p_ref, hw_ref, dis_ref, bg_ref, o_ref):
    dis = dis_ref[...]
    o_ref[...] = (p_ref[0] + p_ref[1]
                  + (2.0 * dis * dis) * hw_ref[...] + bg_ref[...])


def _tc_fin(gcn_p, hw, dis, bg, block_rows=2000):
    grid = (N // block_rows,)
    return pl.pallas_call(
        _fin_body,
        grid=grid,
        in_specs=[
            pl.BlockSpec((NC, block_rows, OUT), lambda i: (0, i, 0)),
            pl.BlockSpec((block_rows, OUT), lambda i: (i, 0)),
            pl.BlockSpec((block_rows, 1), lambda i: (i, 0)),
            pl.BlockSpec((1, OUT), lambda i: (0, 0)),
        ],
        out_specs=pl.BlockSpec((block_rows, OUT), lambda i: (i, 0)),
        out_shape=jax.ShapeDtypeStruct((N, OUT), jnp.float32),
    )(gcn_p, hw, dis, bg)


def _sca_body(esrc, edst, ea, q, k, v, we,
              num_out, ex_out,
              idx_s, idx_d, qr, kr, vr, eab, exb, wev, acc,
              sem1, sem2, sem3):
    c = lax.axis_index("c")
    s = lax.axis_index("s")
    wid = s * NC + c
    rows0 = s * RPS
    lane = lax.broadcasted_iota(jnp.int32, (L,), 0)
    oh = [jnp.where(lane == i, 1.0, 0.0) for i in range(5)]
    mlt4 = lane < 4
    zv = oh[0] * 0.0

    def _zrow(r, c0):
        for p_ in range(F // L):
            vr[r, pl.ds(p_ * L, L)] = zv
        return c0

    lax.fori_loop(0, CH, _zrow, 0)

    def _zcp(t, c0):
        pltpu.sync_copy(vr, acc.at[pl.ds(rows0 + t * CH, CH)])
        return c0

    lax.fori_loop(0, RPS // CH, _zcp, 0)
    pltpu.sync_copy(we, wev)
    plsc.subcore_barrier()

    we00 = wev[0, pl.ds(0, L)]
    we01 = wev[0, pl.ds(L, L)]
    we10 = wev[1, pl.ds(0, L)]
    we11 = wev[1, pl.ds(L, L)]
    base_w = wid * EPW

    def chunk_body(i, carry):
        base = base_w + i * CH
        pltpu.sync_copy(esrc.at[pl.ds(base, CH)], idx_s)
        pltpu.sync_copy(edst.at[pl.ds(base, CH)], idx_d)
        pltpu.sync_copy(ea.at[pl.ds(base * 2, CH * 2)], eab)
        cp1 = pltpu.async_copy(k.at[idx_s], kr, sem1)
        cp2 = pltpu.async_copy(q.at[idx_d], qr, sem2)
        cp3 = pltpu.async_copy(v.at[idx_s], vr, sem3)
        cp1.wait()
        cp2.wait()
        cp3.wait()

        def grp_body(g, carry2):
            eav = eab[pl.ds(g * 2 * L, L)]
            for m in range(8):
                j = g * 8 + m
                a0 = eav[2 * m]
                a1 = eav[2 * m + 1]
                e0 = a0 * we00 + a1 * we10
                e1 = a0 * we01 + a1 * we11
                denrow = a1 * oh[4]
                for b in range(NB):
                    t = (qr[j, pl.ds(b * 2 * L, L)]
                         * (kr[j, pl.ds(b * 2 * L, L)] + e0)
                         + qr[j, pl.ds(b * 2 * L + L, L)]
                         * (kr[j, pl.ds(b * 2 * L + L, L)] + e1))
                    alpha = jnp.sum(t) * _INV_SQRT
                    denrow = denrow + alpha * oh[b]
                exrow = jnp.exp(denrow)
                denrow = jnp.where(mlt4, exrow, denrow)
                exb[pl.ds(j * L, L)] = denrow
                for b in range(NB):
                    exs = exrow[b]
                    vr[j, pl.ds(b * 2 * L, L)] = (
                        vr[j, pl.ds(b * 2 * L, L)] + e0) * exs
                    vr[j, pl.ds(b * 2 * L + L, L)] = (
                        vr[j, pl.ds(b * 2 * L + L, L)] + e1) * exs
            return carry2

        lax.fori_loop(0, CH // 8, grp_body, 0)
        pltpu.sync_copy(vr, acc.at[idx_d], add=True)
        pltpu.sync_copy(exb, ex_out.at[pl.ds(base * L, CH * L)])
        return carry

    lax.fori_loop(0, NCHUNK, chunk_body, 0)
    plsc.subcore_barrier()
    pltpu.sync_copy(acc.at[pl.ds(rows0, RPS)],
                    num_out.at[c, pl.ds(rows0, RPS)])


def _sc_pass_a(esrc, edst, ea, q, k, v, we):
    f = pl.kernel(
        _sca_body,
        out_type=[
            jax.ShapeDtypeStruct((NC, NACC, F), jnp.float32),
            jax.ShapeDtypeStruct((E * L,), jnp.float32),
        ],
        mesh=_mesh(),
        scratch_types=[
            pltpu.VMEM((CH,), jnp.int32),
            pltpu.VMEM((CH,), jnp.int32),
            pltpu.VMEM((CH, F), jnp.float32),
            pltpu.VMEM((CH, F), jnp.float32),
            pltpu.VMEM((CH, F), jnp.float32),
            pltpu.VMEM((CH * 2,), jnp.float32),
            pltpu.VMEM((CH * L,), jnp.float32),
            pltpu.VMEM((2, HC), jnp.float32),
            pltpu.VMEM_SHARED((NACC, F), jnp.float32),
            pltpu.SemaphoreType.DMA,
            pltpu.SemaphoreType.DMA,
            pltpu.SemaphoreType.DMA,
        ],
        compiler_params=pltpu.CompilerParams(needs_layout_passes=False),
    )
    return f(esrc, edst, ea, q, k, v, we)


def _den_body(edst, exf, den_out, idx_d, denb, den_acc):
    c = lax.axis_index("c")
    s = lax.axis_index("s")
    wid = s * NC + c
    rows0 = s * RPS
    lane = lax.broadcasted_iota(jnp.int32, (L,), 0)
    zv = jnp.where(lane == 0, 0.0, 0.0)

    def _zrow(r, c0):
        denb[r, :] = zv
        return c0

    lax.fori_loop(0, CH, _zrow, 0)

    def _zcp(t, c0):
        pltpu.sync_copy(denb, den_acc.at[pl.ds(rows0 + t * CH, CH)])
        return c0

    lax.fori_loop(0, RPS // CH, _zcp, 0)
    plsc.subcore_barrier()
    base_w = wid * EPW

    def chunk_body(i, carry):
        base = base_w + i * CH
        pltpu.sync_copy(edst.at[pl.ds(base, CH)], idx_d)
        pltpu.sync_copy(exf.at[pl.ds(base, CH)], denb)
        pltpu.sync_copy(denb, den_acc.at[idx_d], add=True)
        return carry

    lax.fori_loop(0, NCHUNK, chunk_body, 0)
    plsc.subcore_barrier()
    pltpu.sync_copy(den_acc.at[pl.ds(rows0, RPS)],
                    den_out.at[c, pl.ds(rows0, RPS)])


def _sc_pass_den(edst, exf):
    f = pl.kernel(
        _den_body,
        out_type=jax.ShapeDtypeStruct((NC, NACC, 16), jnp.float32),
        mesh=_mesh(),
        scratch_types=[
            pltpu.VMEM((CH,), jnp.int32),
            pltpu.VMEM((CH, 16), jnp.float32),
            pltpu.VMEM_SHARED((NACC, 16), jnp.float32),
        ],
        compiler_params=pltpu.CompilerParams(needs_layout_passes=False),
    )
    return f(edst, exf)


def _scb_body(esrc, edst, ea, hw, dis, zout,
              gcn_out,
              idx_s, idx_d, hr, eab, disb, acc, sem):
    c = lax.axis_index("c")
    s = lax.axis_index("s")
    wid = s * NC + c
    rows0 = s * RPS
    pltpu.sync_copy(zout.at[pl.ds(rows0, RPS)], acc.at[pl.ds(rows0, RPS)])
    pltpu.sync_copy(dis, disb)
    plsc.subcore_barrier()

    base_w = wid * EPW

    def chunk_body(i, carry):
        base = base_w + i * CH
        pltpu.sync_copy(esrc.at[pl.ds(base, CH)], idx_s)
        pltpu.sync_copy(edst.at[pl.ds(base, CH)], idx_d)
        pltpu.sync_copy(ea.at[pl.ds(base * 2, CH * 2)], eab)
        cp = pltpu.async_copy(hw.at[idx_s], hr, sem)
        cp.wait()

        def grp_body(g, carry2):
            srcv = idx_s[pl.ds(g * L, L)]
            dstv = idx_d[pl.ds(g * L, L)]
            ridx = lax.broadcasted_iota(jnp.int32, (L,), 0) + g * L
            ewidx = ridx * 2 + 1
            ew = plsc.load_gather(eab, [ewidx])
            dsrc = plsc.load_gather(disb, [srcv])
            ddst = plsc.load_gather(disb, [dstv])
            wv = dsrc * ew * ddst
            for m in range(L):
                j = g * L + m
                wj = wv[m]
                for p in range(F // L):
                    hr[j, pl.ds(p * L, L)] = hr[j, pl.ds(p * L, L)] * wj
            return carry2

        lax.fori_loop(0, CH // L, grp_body, 0)
        pltpu.sync_copy(hr, acc.at[idx_d], add=True)
        return carry

    lax.fori_loop(0, NCHUNK, chunk_body, 0)
    plsc.subcore_barrier()
    pltpu.sync_copy(acc.at[pl.ds(rows0, RPS)],
                    gcn_out.at[c, pl.ds(rows0, RPS)])


def _sc_pass_b(esrc, edst, ea, hw, dis, zout):
    f = pl.kernel(
        _scb_body,
        out_type=jax.ShapeDtypeStruct((NC, NACC, OUT), jnp.float32),
        mesh=_mesh(),
        scratch_types=[
            pltpu.VMEM((CH,), jnp.int32),
            pltpu.VMEM((CH,), jnp.int32),
            pltpu.VMEM((CH, OUT), jnp.float32),
            pltpu.VMEM((CH * 2,), jnp.float32),
            pltpu.VMEM((N,), jnp.float32),
            pltpu.VMEM_SHARED((NACC, OUT), jnp.float32),
            pltpu.SemaphoreType.DMA,
        ],
        compiler_params=pltpu.CompilerParams(needs_layout_passes=False),
    )
    return f(esrc, edst, ea, hw, dis, zout)


def _block_diag(w):
    out = jnp.zeros((F, NB * HC), dtype=w.dtype)
    for i in range(NB):
        out = out.at[i * IC:(i + 1) * IC, i * HC:(i + 1) * HC].set(w)
    return out


def kernel(x, edge_index, edge_attrs, Wq, bq, Wk, bk, Wv, bv, We, Ws, bs, Wg, bg):
    wcat = jnp.concatenate(
        [_block_diag(Wq), _block_diag(Wk), _block_diag(Wv), _block_diag(Ws)],
        axis=1)
    bcat = jnp.concatenate(
        [jnp.tile(bq, NB), jnp.tile(bk, NB), jnp.tile(bv, NB),
         jnp.tile(bs, NB)])[None, :]
    q, k, v, s = _tc_proj(x, wcat, bcat)

    esrc = edge_index[0].astype(jnp.int32)
    edst = edge_index[1].astype(jnp.int32)
    eaf = edge_attrs.reshape(E * 2)
    znum = jnp.zeros((NACC, F), jnp.float32)
    num_p, exf = _sc_pass_a(esrc, edst, eaf, q, k, v, We)
    den_p = _sc_pass_den(edst, exf.reshape(E, L))

    hw, dis = _tc_mid(num_p, den_p, s, Wg)
    gcn_p = _sc_pass_b(esrc, edst, eaf, hw, dis.reshape(N), znum)
    return _tc_fin(gcn_p, hw, dis, bg[None, :])
